# baseline (speedup 1.0000x reference)
import functools

import jax
import jax.numpy as jnp
from jax import lax
from jax.experimental import pallas as pl
from jax.experimental.pallas import tpu as pltpu
from jax.experimental.pallas import tpu_sc as plsc

NUM_CODES = 8192
DIM = 32
NPIX = 8192
PB = 256
CK = 2048
NCHUNK = NUM_CODES // CK
COMMIT = 0.25

NWORKERS = 32
BPW = NPIX // NWORKERS
IDX_CHUNK = 128


def _argmin_body(z_ref, z2_ref, cb_ref, c2_ref, lanef_ref, idx_ref, d_ref):
    z = z_ref[...]
    z2 = z2_ref[...]
    ms = []
    for j in range(NCHUNK):
        cb = cb_ref[j * CK:(j + 1) * CK, :]
        c2 = c2_ref[:, j * CK:(j + 1) * CK]
        s = lax.dot_general(z, cb, (((1,), (1,)), ((), ())),
                            preferred_element_type=jnp.float32)
        d = (z2 + c2) + s * jnp.float32(-2.0)
        d_ref[:, j * CK:(j + 1) * CK] = d
        ms.append(jnp.min(d, axis=1, keepdims=True))
    best = jnp.full((PB, 1), jnp.inf, dtype=jnp.float32)
    tsel = jnp.zeros((PB, 1), dtype=jnp.int32)
    for j in range(NCHUNK):
        upd = best > ms[j]
        best = jnp.where(upd, ms[j], best).astype(jnp.bfloat16).astype(jnp.float32)
        tsel = jnp.where(upd, j, tsel)
    mstar = jnp.where(tsel == 0, ms[0],
                      jnp.where(tsel == 1, ms[1],
                                jnp.where(tsel == 2, ms[2], ms[3])))
    d01 = jnp.where(tsel == 0, d_ref[:, 0:CK], d_ref[:, CK:2 * CK])
    d23 = jnp.where(tsel == 2, d_ref[:, 2 * CK:3 * CK], d_ref[:, 3 * CK:4 * CK])
    dw = jnp.where(tsel <= 1, d01, d23)
    lanef = lanef_ref[...]
    lif = jnp.min(jnp.where(dw == mstar, lanef, jnp.float32(1e9)),
                  axis=1, keepdims=True)
    idx_ref[...] = lif.astype(jnp.int32) + tsel * CK


def _argmin_call(z_flat, z2, codebook, c2row):
    return pl.pallas_call(
        _argmin_body,
        grid=(NPIX // PB,),
        in_specs=[
            pl.BlockSpec((PB, DIM), lambda i: (i, 0)),
            pl.BlockSpec((PB, 1), lambda i: (i, 0)),
            pl.BlockSpec((NUM_CODES, DIM), lambda i: (0, 0)),
            pl.BlockSpec((1, NUM_CODES), lambda i: (0, 0)),
            pl.BlockSpec((1, CK), lambda i: (0, 0)),
        ],
        out_specs=pl.BlockSpec((PB, 1), lambda i: (i, 0)),
        out_shape=jax.ShapeDtypeStruct((NPIX, 1), jnp.int32),
        scratch_shapes=[pltpu.VMEM((PB, NUM_CODES), jnp.float32)],
    )(z_flat, z2, codebook, c2row,
      lax.broadcasted_iota(jnp.float32, (1, CK), 1))


def _sc_body(cb_hbm, idx_hbm, z_hbm, zq_hbm, parts_hbm,
             idx_v, rows_v, z_v, acc_v, sem):
    wid = lax.axis_index("s") * 2 + lax.axis_index("c")
    base = wid * BPW
    pltpu.sync_copy(idx_hbm.at[pl.ds(wid * 2, 2)], idx_v)
    cp0 = pltpu.async_copy(cb_hbm.at[idx_v.at[0]],
                           rows_v.at[pl.ds(0, IDX_CHUNK)], sem)
    cp1 = pltpu.async_copy(cb_hbm.at[idx_v.at[1]],
                           rows_v.at[pl.ds(IDX_CHUNK, IDX_CHUNK)], sem)
    pltpu.sync_copy(z_hbm.at[pl.ds(base, BPW)], z_v)
    cp0.wait()
    cp1.wait()

    def body(p, acc):
        r0 = rows_v[p, 0:16]
        r1 = rows_v[p, 16:32]
        x0 = z_v[p, 0:16]
        x1 = z_v[p, 16:32]
        d0 = r0 - x0
        d1 = r1 - x1
        rows_v[p, 0:16] = x0 + d0
        rows_v[p, 16:32] = x1 + d1
        return acc + d0 * d0 + d1 * d1

    acc = lax.fori_loop(0, BPW, body, jnp.zeros((16,), jnp.float32))
    acc_v[...] = acc
    pltpu.sync_copy(rows_v, zq_hbm.at[pl.ds(base, BPW)])
    pltpu.sync_copy(acc_v, parts_hbm.at[wid])


def _sc_call(codebook, idx2d, z_flat):
    fn = pl.kernel(
        _sc_body,
        out_type=(jax.ShapeDtypeStruct((NPIX, DIM), jnp.float32),
                  jax.ShapeDtypeStruct((NWORKERS, 16), jnp.float32)),
        mesh=plsc.VectorSubcoreMesh(core_axis_name="c", subcore_axis_name="s"),
        compiler_params=pltpu.CompilerParams(use_tc_tiling_on_sc=False),
        scratch_types=[
            pltpu.VMEM((2, IDX_CHUNK), jnp.int32),
            pltpu.VMEM((BPW, DIM), jnp.float32),
            pltpu.VMEM((BPW, DIM), jnp.float32),
            pltpu.VMEM((16,), jnp.float32),
            pltpu.SemaphoreType.DMA,
        ],
    )
    return fn(codebook, idx2d, z_flat)


def kernel(z, codebook):
    zt = jnp.transpose(z, (0, 2, 3, 1))
    z_flat = zt.reshape(-1, DIM)
    z2 = jnp.sum(z_flat ** 2, axis=1, keepdims=True)
    c2row = jnp.sum(codebook ** 2, axis=1)[None, :]
    idx = _argmin_call(z_flat, z2, codebook, c2row)
    zq_out_flat, parts = _sc_call(codebook, idx.reshape(64, 128), z_flat)
    v = jnp.sum(parts) / (NPIX * DIM)
    loss = v + COMMIT * v
    z_q_out = jnp.transpose(zq_out_flat.reshape(8, 32, 32, DIM), (0, 3, 1, 2))
    indices_out = idx.reshape(8, 32, 32)
    return (z_q_out, loss, indices_out)


# speedup vs baseline: 1.0381x; 1.0381x over previous
import functools

import jax
import jax.numpy as jnp
from jax import lax
from jax.experimental import pallas as pl
from jax.experimental.pallas import tpu as pltpu
from jax.experimental.pallas import tpu_sc as plsc

NUM_CODES = 8192
DIM = 32
NPIX = 8192
PB = 256
CK = 2048
NCHUNK = NUM_CODES // CK
COMMIT = 0.25

NWORKERS = 32
BPW = NPIX // NWORKERS
IDX_CHUNK = 128


def _argmin_body(z_ref, z2_ref, cb_ref, c2_ref, lanef_ref, idx_ref, zf_ref,
                 d_ref):
    zt = z_ref[0]
    z2 = z2_ref[...]
    zf_ref[...] = jnp.transpose(zt, (1, 0))
    ms = []
    for j in range(NCHUNK):
        cb = cb_ref[j * CK:(j + 1) * CK, :]
        c2 = c2_ref[:, j * CK:(j + 1) * CK]
        s = lax.dot_general(zt, cb, (((0,), (1,)), ((), ())),
                            preferred_element_type=jnp.float32)
        d = (z2 + c2) + s * jnp.float32(-2.0)
        d_ref[:, j * CK:(j + 1) * CK] = d
        ms.append(jnp.min(d, axis=1, keepdims=True))
    best = jnp.full((PB, 1), jnp.inf, dtype=jnp.float32)
    tsel = jnp.zeros((PB, 1), dtype=jnp.int32)
    for j in range(NCHUNK):
        upd = best > ms[j]
        best = jnp.where(upd, ms[j], best).astype(jnp.bfloat16).astype(jnp.float32)
        tsel = jnp.where(upd, j, tsel)
    mstar = jnp.where(tsel == 0, ms[0],
                      jnp.where(tsel == 1, ms[1],
                                jnp.where(tsel == 2, ms[2], ms[3])))
    d01 = jnp.where(tsel == 0, d_ref[:, 0:CK], d_ref[:, CK:2 * CK])
    d23 = jnp.where(tsel == 2, d_ref[:, 2 * CK:3 * CK], d_ref[:, 3 * CK:4 * CK])
    dw = jnp.where(tsel <= 1, d01, d23)
    lanef = lanef_ref[...]
    lif = jnp.min(jnp.where(dw == mstar, lanef, jnp.float32(1e9)),
                  axis=1, keepdims=True)
    idx_ref[...] = lif.astype(jnp.int32) + tsel * CK


def _argmin_call(z3, z2, codebook, c2row):
    return pl.pallas_call(
        _argmin_body,
        grid=(NPIX // PB,),
        in_specs=[
            pl.BlockSpec((1, DIM, PB), lambda i: (i // 4, 0, i % 4)),
            pl.BlockSpec((PB, 1), lambda i: (i, 0)),
            pl.BlockSpec((NUM_CODES, DIM), lambda i: (0, 0)),
            pl.BlockSpec((1, NUM_CODES), lambda i: (0, 0)),
            pl.BlockSpec((1, CK), lambda i: (0, 0)),
        ],
        out_specs=[pl.BlockSpec((PB, 1), lambda i: (i, 0)),
                   pl.BlockSpec((PB, DIM), lambda i: (i, 0))],
        out_shape=[jax.ShapeDtypeStruct((NPIX, 1), jnp.int32),
                   jax.ShapeDtypeStruct((NPIX, DIM), jnp.float32)],
        scratch_shapes=[pltpu.VMEM((PB, NUM_CODES), jnp.float32)],
    )(z3, z2, codebook, c2row,
      lax.broadcasted_iota(jnp.float32, (1, CK), 1))


def _sc_body(cb_hbm, idx_hbm, z_hbm, zq_hbm, parts_hbm,
             idx_v, rows_v, z_v, acc_v, sem):
    wid = lax.axis_index("s") * 2 + lax.axis_index("c")
    base = wid * BPW
    pltpu.sync_copy(idx_hbm.at[pl.ds(wid * 2, 2)], idx_v)
    cp0 = pltpu.async_copy(cb_hbm.at[idx_v.at[0]],
                           rows_v.at[pl.ds(0, IDX_CHUNK)], sem)
    cp1 = pltpu.async_copy(cb_hbm.at[idx_v.at[1]],
                           rows_v.at[pl.ds(IDX_CHUNK, IDX_CHUNK)], sem)
    pltpu.sync_copy(z_hbm.at[pl.ds(base, BPW)], z_v)
    cp0.wait()
    cp1.wait()

    def body(p, acc):
        r0 = rows_v[p, 0:16]
        r1 = rows_v[p, 16:32]
        x0 = z_v[p, 0:16]
        x1 = z_v[p, 16:32]
        d0 = r0 - x0
        d1 = r1 - x1
        rows_v[p, 0:16] = x0 + d0
        rows_v[p, 16:32] = x1 + d1
        return acc + d0 * d0 + d1 * d1

    acc = lax.fori_loop(0, BPW, body, jnp.zeros((16,), jnp.float32))
    acc_v[...] = acc
    pltpu.sync_copy(rows_v, zq_hbm.at[pl.ds(base, BPW)])
    pltpu.sync_copy(acc_v, parts_hbm.at[wid])


def _sc_call(codebook, idx2d, z_flat):
    fn = pl.kernel(
        _sc_body,
        out_type=(jax.ShapeDtypeStruct((NPIX, DIM), jnp.float32),
                  jax.ShapeDtypeStruct((NWORKERS, 16), jnp.float32)),
        mesh=plsc.VectorSubcoreMesh(core_axis_name="c", subcore_axis_name="s"),
        compiler_params=pltpu.CompilerParams(use_tc_tiling_on_sc=False),
        scratch_types=[
            pltpu.VMEM((2, IDX_CHUNK), jnp.int32),
            pltpu.VMEM((BPW, DIM), jnp.float32),
            pltpu.VMEM((BPW, DIM), jnp.float32),
            pltpu.VMEM((16,), jnp.float32),
            pltpu.SemaphoreType.DMA,
        ],
    )
    return fn(codebook, idx2d, z_flat)


def kernel(z, codebook):
    zt = jnp.transpose(z, (0, 2, 3, 1))
    z2 = jnp.sum(zt.reshape(-1, DIM) ** 2, axis=1, keepdims=True)
    c2row = jnp.sum(codebook ** 2, axis=1)[None, :]
    idx, z_flat = _argmin_call(z.reshape(8, DIM, 1024), z2, codebook, c2row)
    zq_out_flat, parts = _sc_call(codebook, idx.reshape(64, 128), z_flat)
    v = jnp.sum(parts) / (NPIX * DIM)
    loss = v + COMMIT * v
    z_q_out = jnp.transpose(zq_out_flat.reshape(8, 32, 32, DIM), (0, 3, 1, 2))
    indices_out = idx.reshape(8, 32, 32)
    return (z_q_out, loss, indices_out)


# speedup vs baseline: 1.0932x; 1.0531x over previous
import functools

import jax
import jax.numpy as jnp
from jax import lax
from jax.experimental import pallas as pl
from jax.experimental.pallas import tpu as pltpu
from jax.experimental.pallas import tpu_sc as plsc

NUM_CODES = 8192
DIM = 32
NPIX = 8192
PB = 512
CK = 2048
NCHUNK = NUM_CODES // CK
COMMIT = 0.25

NWORKERS = 32
BPW = NPIX // NWORKERS
IDX_CHUNK = 128


def _argmin_body(z_ref, z2_ref, cb_ref, c2_ref, lanef_ref, idx_ref, zf_ref,
                 d_ref):
    zt = z_ref[0]
    z2 = z2_ref[...]
    zf_ref[...] = jnp.transpose(zt, (1, 0))
    ms = []
    for j in range(NCHUNK):
        cb = cb_ref[j * CK:(j + 1) * CK, :]
        c2 = c2_ref[:, j * CK:(j + 1) * CK]
        s = lax.dot_general(zt, cb, (((0,), (1,)), ((), ())),
                            preferred_element_type=jnp.float32)
        d = (z2 + c2) + s * jnp.float32(-2.0)
        d_ref[:, j * CK:(j + 1) * CK] = d
        ms.append(jnp.min(d, axis=1, keepdims=True))
    best = jnp.full((PB, 1), jnp.inf, dtype=jnp.float32)
    tsel = jnp.zeros((PB, 1), dtype=jnp.int32)
    for j in range(NCHUNK):
        upd = best > ms[j]
        best = jnp.where(upd, ms[j], best).astype(jnp.bfloat16).astype(jnp.float32)
        tsel = jnp.where(upd, j, tsel)
    mstar = jnp.where(tsel == 0, ms[0],
                      jnp.where(tsel == 1, ms[1],
                                jnp.where(tsel == 2, ms[2], ms[3])))
    d01 = jnp.where(tsel == 0, d_ref[:, 0:CK], d_ref[:, CK:2 * CK])
    d23 = jnp.where(tsel == 2, d_ref[:, 2 * CK:3 * CK], d_ref[:, 3 * CK:4 * CK])
    dw = jnp.where(tsel <= 1, d01, d23)
    lanef = lanef_ref[...]
    lif = jnp.min(jnp.where(dw == mstar, lanef, jnp.float32(1e9)),
                  axis=1, keepdims=True)
    idx_ref[...] = lif.astype(jnp.int32) + tsel * CK


def _argmin_call(z3, z2, codebook, c2row):
    return pl.pallas_call(
        _argmin_body,
        grid=(NPIX // PB,),
        in_specs=[
            pl.BlockSpec((1, DIM, PB), lambda i: (i // 2, 0, i % 2)),
            pl.BlockSpec((PB, 1), lambda i: (i, 0)),
            pl.BlockSpec((NUM_CODES, DIM), lambda i: (0, 0)),
            pl.BlockSpec((1, NUM_CODES), lambda i: (0, 0)),
            pl.BlockSpec((1, CK), lambda i: (0, 0)),
        ],
        out_specs=[pl.BlockSpec((PB, 1), lambda i: (i, 0)),
                   pl.BlockSpec((PB, DIM), lambda i: (i, 0))],
        out_shape=[jax.ShapeDtypeStruct((NPIX, 1), jnp.int32),
                   jax.ShapeDtypeStruct((NPIX, DIM), jnp.float32)],
        scratch_shapes=[pltpu.VMEM((PB, NUM_CODES), jnp.float32)],
    )(z3, z2, codebook, c2row,
      lax.broadcasted_iota(jnp.float32, (1, CK), 1))


def _sc_body(cb_hbm, idx_hbm, z_hbm, zq_hbm, parts_hbm,
             idx_v, rows_v, z_v, acc_v, sem):
    wid = lax.axis_index("s") * 2 + lax.axis_index("c")
    base = wid * BPW
    pltpu.sync_copy(idx_hbm.at[pl.ds(wid * 2, 2)], idx_v)
    cp0 = pltpu.async_copy(cb_hbm.at[idx_v.at[0]],
                           rows_v.at[pl.ds(0, IDX_CHUNK)], sem)
    cp1 = pltpu.async_copy(cb_hbm.at[idx_v.at[1]],
                           rows_v.at[pl.ds(IDX_CHUNK, IDX_CHUNK)], sem)
    pltpu.sync_copy(z_hbm.at[pl.ds(base, BPW)], z_v)
    cp0.wait()
    cp1.wait()

    def body(p, acc):
        r0 = rows_v[p, 0:16]
        r1 = rows_v[p, 16:32]
        x0 = z_v[p, 0:16]
        x1 = z_v[p, 16:32]
        d0 = r0 - x0
        d1 = r1 - x1
        rows_v[p, 0:16] = x0 + d0
        rows_v[p, 16:32] = x1 + d1
        return acc + d0 * d0 + d1 * d1

    acc = lax.fori_loop(0, BPW, body, jnp.zeros((16,), jnp.float32))
    acc_v[...] = acc
    pltpu.sync_copy(rows_v, zq_hbm.at[pl.ds(base, BPW)])
    pltpu.sync_copy(acc_v, parts_hbm.at[wid])


def _sc_call(codebook, idx2d, z_flat):
    fn = pl.kernel(
        _sc_body,
        out_type=(jax.ShapeDtypeStruct((NPIX, DIM), jnp.float32),
                  jax.ShapeDtypeStruct((NWORKERS, 16), jnp.float32)),
        mesh=plsc.VectorSubcoreMesh(core_axis_name="c", subcore_axis_name="s"),
        compiler_params=pltpu.CompilerParams(use_tc_tiling_on_sc=False),
        scratch_types=[
            pltpu.VMEM((2, IDX_CHUNK), jnp.int32),
            pltpu.VMEM((BPW, DIM), jnp.float32),
            pltpu.VMEM((BPW, DIM), jnp.float32),
            pltpu.VMEM((16,), jnp.float32),
            pltpu.SemaphoreType.DMA,
        ],
    )
    return fn(codebook, idx2d, z_flat)


def kernel(z, codebook):
    zt = jnp.transpose(z, (0, 2, 3, 1))
    z2 = jnp.sum(zt.reshape(-1, DIM) ** 2, axis=1, keepdims=True)
    c2row = jnp.sum(codebook ** 2, axis=1)[None, :]
    idx, z_flat = _argmin_call(z.reshape(8, DIM, 1024), z2, codebook, c2row)
    zq_out_flat, parts = _sc_call(codebook, idx.reshape(64, 128), z_flat)
    v = jnp.sum(parts) / (NPIX * DIM)
    loss = v + COMMIT * v
    z_q_out = jnp.transpose(zq_out_flat.reshape(8, 32, 32, DIM), (0, 3, 1, 2))
    indices_out = idx.reshape(8, 32, 32)
    return (z_q_out, loss, indices_out)


# speedup vs baseline: 1.1195x; 1.0240x over previous
import functools

import jax
import jax.numpy as jnp
from jax import lax
from jax.experimental import pallas as pl
from jax.experimental.pallas import tpu as pltpu
from jax.experimental.pallas import tpu_sc as plsc

NUM_CODES = 8192
DIM = 32
NPIX = 8192
PB = 1024
CK = 2048
NCHUNK = NUM_CODES // CK
COMMIT = 0.25

NWORKERS = 32
BPW = NPIX // NWORKERS
IDX_CHUNK = 128


def _argmin_body(z_ref, z2_ref, cb_ref, c2_ref, lanef_ref, idx_ref, zf_ref,
                 d_ref):
    zt = z_ref[0]
    z2 = z2_ref[...]
    zf_ref[...] = jnp.transpose(zt, (1, 0))
    ms = []
    for j in range(NCHUNK):
        cb = cb_ref[j * CK:(j + 1) * CK, :]
        c2 = c2_ref[:, j * CK:(j + 1) * CK]
        s = lax.dot_general(zt, cb, (((0,), (1,)), ((), ())),
                            preferred_element_type=jnp.float32)
        d = (z2 + c2) + s * jnp.float32(-2.0)
        d_ref[:, j * CK:(j + 1) * CK] = d
        ms.append(jnp.min(d, axis=1, keepdims=True))
    best = jnp.full((PB, 1), jnp.inf, dtype=jnp.float32)
    tsel = jnp.zeros((PB, 1), dtype=jnp.int32)
    for j in range(NCHUNK):
        upd = best > ms[j]
        best = jnp.where(upd, ms[j], best).astype(jnp.bfloat16).astype(jnp.float32)
        tsel = jnp.where(upd, j, tsel)
    mstar = jnp.where(tsel == 0, ms[0],
                      jnp.where(tsel == 1, ms[1],
                                jnp.where(tsel == 2, ms[2], ms[3])))
    d01 = jnp.where(tsel == 0, d_ref[:, 0:CK], d_ref[:, CK:2 * CK])
    d23 = jnp.where(tsel == 2, d_ref[:, 2 * CK:3 * CK], d_ref[:, 3 * CK:4 * CK])
    dw = jnp.where(tsel <= 1, d01, d23)
    lanef = lanef_ref[...]
    lif = jnp.min(jnp.where(dw == mstar, lanef, jnp.float32(1e9)),
                  axis=1, keepdims=True)
    idx_ref[...] = lif.astype(jnp.int32) + tsel * CK


def _argmin_call(z3, z2, codebook, c2row):
    return pl.pallas_call(
        _argmin_body,
        grid=(NPIX // PB,),
        in_specs=[
            pl.BlockSpec((1, DIM, PB), lambda i: (i, 0, 0)),
            pl.BlockSpec((PB, 1), lambda i: (i, 0)),
            pl.BlockSpec((NUM_CODES, DIM), lambda i: (0, 0)),
            pl.BlockSpec((1, NUM_CODES), lambda i: (0, 0)),
            pl.BlockSpec((1, CK), lambda i: (0, 0)),
        ],
        out_specs=[pl.BlockSpec((PB, 1), lambda i: (i, 0)),
                   pl.BlockSpec((PB, DIM), lambda i: (i, 0))],
        out_shape=[jax.ShapeDtypeStruct((NPIX, 1), jnp.int32),
                   jax.ShapeDtypeStruct((NPIX, DIM), jnp.float32)],
        scratch_shapes=[pltpu.VMEM((PB, NUM_CODES), jnp.float32)],
    )(z3, z2, codebook, c2row,
      lax.broadcasted_iota(jnp.float32, (1, CK), 1))


def _sc_body(cb_hbm, idx_hbm, z_hbm, zq_hbm, parts_hbm,
             idx_v, rows_v, z_v, acc_v, sem):
    wid = lax.axis_index("s") * 2 + lax.axis_index("c")
    base = wid * BPW
    pltpu.sync_copy(idx_hbm.at[pl.ds(wid * 2, 2)], idx_v)
    cp0 = pltpu.async_copy(cb_hbm.at[idx_v.at[0]],
                           rows_v.at[pl.ds(0, IDX_CHUNK)], sem)
    cp1 = pltpu.async_copy(cb_hbm.at[idx_v.at[1]],
                           rows_v.at[pl.ds(IDX_CHUNK, IDX_CHUNK)], sem)
    pltpu.sync_copy(z_hbm.at[pl.ds(base, BPW)], z_v)
    cp0.wait()
    cp1.wait()

    def body(p, acc):
        r0 = rows_v[p, 0:16]
        r1 = rows_v[p, 16:32]
        x0 = z_v[p, 0:16]
        x1 = z_v[p, 16:32]
        d0 = r0 - x0
        d1 = r1 - x1
        rows_v[p, 0:16] = x0 + d0
        rows_v[p, 16:32] = x1 + d1
        return acc + d0 * d0 + d1 * d1

    acc = lax.fori_loop(0, BPW, body, jnp.zeros((16,), jnp.float32))
    acc_v[...] = acc
    pltpu.sync_copy(rows_v, zq_hbm.at[pl.ds(base, BPW)])
    pltpu.sync_copy(acc_v, parts_hbm.at[wid])


def _sc_call(codebook, idx2d, z_flat):
    fn = pl.kernel(
        _sc_body,
        out_type=(jax.ShapeDtypeStruct((NPIX, DIM), jnp.float32),
                  jax.ShapeDtypeStruct((NWORKERS, 16), jnp.float32)),
        mesh=plsc.VectorSubcoreMesh(core_axis_name="c", subcore_axis_name="s"),
        compiler_params=pltpu.CompilerParams(use_tc_tiling_on_sc=False),
        scratch_types=[
            pltpu.VMEM((2, IDX_CHUNK), jnp.int32),
            pltpu.VMEM((BPW, DIM), jnp.float32),
            pltpu.VMEM((BPW, DIM), jnp.float32),
            pltpu.VMEM((16,), jnp.float32),
            pltpu.SemaphoreType.DMA,
        ],
    )
    return fn(codebook, idx2d, z_flat)


def kernel(z, codebook):
    zt = jnp.transpose(z, (0, 2, 3, 1))
    z2 = jnp.sum(zt.reshape(-1, DIM) ** 2, axis=1, keepdims=True)
    c2row = jnp.sum(codebook ** 2, axis=1)[None, :]
    idx, z_flat = _argmin_call(z.reshape(8, DIM, 1024), z2, codebook, c2row)
    zq_out_flat, parts = _sc_call(codebook, idx.reshape(64, 128), z_flat)
    v = jnp.sum(parts) / (NPIX * DIM)
    loss = v + COMMIT * v
    z_q_out = jnp.transpose(zq_out_flat.reshape(8, 32, 32, DIM), (0, 3, 1, 2))
    indices_out = idx.reshape(8, 32, 32)
    return (z_q_out, loss, indices_out)
